# baseline (speedup 1.0000x reference)
import functools

import jax
import jax.numpy as jnp
from jax import lax
from jax.experimental import pallas as pl
from jax.experimental.pallas import tpu as pltpu
from jax.experimental.pallas import tpu_sc as plsc

_B, _T, _H = 16384, 200, 128
_N = _B * _T
_V = 20
_NC, _NS = 2, 16
_NW = _NC * _NS
_PER_W = _N // _NW
_SUB = 128
_NBUF = 4
_CHUNK = _SUB * _NBUF
_NGRP = _PER_W // _CHUNK
_IDXROWS = _PER_W // _SUB


def _emb_lookup(x2d, emb_weight):
  mesh = plsc.VectorSubcoreMesh(core_axis_name="c", subcore_axis_name="s")

  @functools.partial(
      pl.kernel,
      mesh=mesh,
      out_type=jax.ShapeDtypeStruct((_N, _H), jnp.float32),
      scratch_types=[
          pltpu.VMEM_SHARED((_V, _H), jnp.float32),
          pltpu.VMEM((2, _NBUF, _SUB), jnp.int32),
          pltpu.VMEM((_CHUNK, _H), jnp.float32),
          pltpu.SemaphoreType.DMA,
          pltpu.SemaphoreType.DMA,
          pltpu.SemaphoreType.DMA,
          pltpu.SemaphoreType.DMA,
          pltpu.SemaphoreType.DMA,
          pltpu.SemaphoreType.DMA,
          pltpu.SemaphoreType.DMA,
          pltpu.SemaphoreType.DMA,
          pltpu.SemaphoreType.DMA,
          pltpu.SemaphoreType.DMA,
      ],
  )
  def body(x_hbm, tbl_hbm, out_hbm, tbl_v, idx_v, rows_v,
           g0, g1, g2, g3, s0, s1, s2, s3, i0, i1):
    gsem = (g0, g1, g2, g3)
    ssem = (s0, s1, s2, s3)
    isem = (i0, i1)
    c = lax.axis_index("c")
    s = lax.axis_index("s")
    wid = s * _NC + c
    idxrow0 = wid * _IDXROWS
    outrow0 = wid * _PER_W

    @pl.when(s == 0)
    def _stage():
      pltpu.sync_copy(tbl_hbm, tbl_v)
    plsc.subcore_barrier()

    def idx_src(g):
      off = jnp.minimum(g * _NBUF, _IDXROWS - _NBUF)
      return x_hbm.at[pl.ds(idxrow0 + off, _NBUF)]

    def fire_idx(g, p):
      pltpu.async_copy(idx_src(g), idx_v.at[p], isem[p])

    def wait_idx(g, p):
      pltpu.make_async_copy(idx_src(g), idx_v.at[p], isem[p]).wait()

    def gather_b(g, b, p):
      pltpu.async_copy(
          tbl_v.at[idx_v.at[p].at[b]],
          rows_v.at[pl.ds(b * _SUB, _SUB)],
          gsem[b],
      )

    def wait_gather_b(g, b, p):
      pltpu.make_async_copy(
          tbl_v.at[idx_v.at[p].at[b]],
          rows_v.at[pl.ds(b * _SUB, _SUB)],
          gsem[b],
      ).wait()

    def scatter_b(g, b):
      pltpu.async_copy(
          rows_v.at[pl.ds(b * _SUB, _SUB)],
          out_hbm.at[pl.ds(outrow0 + g * _CHUNK + b * _SUB, _SUB)],
          ssem[b],
      )

    def wait_scatter_b(g, b):
      pltpu.make_async_copy(
          rows_v.at[pl.ds(b * _SUB, _SUB)],
          out_hbm.at[pl.ds(outrow0 + g * _CHUNK + b * _SUB, _SUB)],
          ssem[b],
      ).wait()

    def do_group(g, p, first):
      wait_idx(g, p)
      for b in range(_NBUF):
        if first is not True:
          wait_scatter_b(g - 1, b)
        gather_b(g, b, p)
      for b in range(_NBUF):
        wait_gather_b(g, b, p)
        scatter_b(g, b)
      if first is not False:
        fire_idx(g + 2, p)
      else:
        @pl.when(g + 2 < _NGRP)
        def _pf():
          fire_idx(g + 2, p)

    fire_idx(0, 0)
    fire_idx(1, 1)
    do_group(0, 0, True)
    do_group(1, 1, "peel")

    def step(i, carry):
      do_group(2 * i, 0, False)
      do_group(2 * i + 1, 1, False)
      return carry

    lax.fori_loop(1, _NGRP // 2, step, 0)
    for b in range(_NBUF):
      wait_scatter_b(_NGRP - 1, b)

  return body(x2d, emb_weight)


_R = 2048
_XR = _R // 128


def _tc_body(x_ref, tbl_ref, o_ref):
  o_ref[...] = jnp.broadcast_to(tbl_ref[0:1, :], (_R, 128))


def _tc_emb(x2d, tblp):
  return pl.pallas_call(
      _tc_body,
      grid=(_N // _R,),
      in_specs=[pl.BlockSpec((_XR, 128), lambda i: (i, 0)),
                pl.BlockSpec((32, 128), lambda i: (0, 0))],
      out_specs=pl.BlockSpec((_R, 128), lambda i: (i, 0)),
      out_shape=jax.ShapeDtypeStruct((_N, 128), jnp.float32),
  )(x2d, tblp)


def kernel(x, emb_weight):
  x2d = x.reshape(_N // _SUB, _SUB).astype(jnp.int32)
  tblp = jnp.zeros((32, 128), jnp.float32).at[:_V].set(emb_weight)
  out = _tc_emb(x2d, tblp)
  return out.reshape(_B, _T, _H)


# speedup vs baseline: 1.6059x; 1.6059x over previous
import functools

import jax
import jax.numpy as jnp
from jax import lax
from jax.experimental import pallas as pl
from jax.experimental.pallas import tpu as pltpu
from jax.experimental.pallas import tpu_sc as plsc

_B, _T, _H = 16384, 200, 128
_N = _B * _T
_V = 20
_NC, _NS = 2, 16
_NW = _NC * _NS
_PER_W = _N // _NW
_SUB = 128
_NBUF = 4
_CHUNK = _SUB * _NBUF
_NGRP = _PER_W // _CHUNK
_IDXROWS = _PER_W // _SUB


def _emb_lookup(x2d, emb_weight):
  mesh = plsc.VectorSubcoreMesh(core_axis_name="c", subcore_axis_name="s")

  @functools.partial(
      pl.kernel,
      mesh=mesh,
      out_type=jax.ShapeDtypeStruct((_N, _H), jnp.float32),
      scratch_types=[
          pltpu.VMEM_SHARED((_V, _H), jnp.float32),
          pltpu.VMEM((2, _NBUF, _SUB), jnp.int32),
          pltpu.VMEM((_CHUNK, _H), jnp.float32),
          pltpu.SemaphoreType.DMA,
          pltpu.SemaphoreType.DMA,
          pltpu.SemaphoreType.DMA,
          pltpu.SemaphoreType.DMA,
          pltpu.SemaphoreType.DMA,
          pltpu.SemaphoreType.DMA,
          pltpu.SemaphoreType.DMA,
          pltpu.SemaphoreType.DMA,
          pltpu.SemaphoreType.DMA,
          pltpu.SemaphoreType.DMA,
      ],
  )
  def body(x_hbm, tbl_hbm, out_hbm, tbl_v, idx_v, rows_v,
           g0, g1, g2, g3, s0, s1, s2, s3, i0, i1):
    gsem = (g0, g1, g2, g3)
    ssem = (s0, s1, s2, s3)
    isem = (i0, i1)
    c = lax.axis_index("c")
    s = lax.axis_index("s")
    wid = s * _NC + c
    idxrow0 = wid * _IDXROWS
    outrow0 = wid * _PER_W

    @pl.when(s == 0)
    def _stage():
      pltpu.sync_copy(tbl_hbm, tbl_v)
    plsc.subcore_barrier()

    def idx_src(g):
      off = jnp.minimum(g * _NBUF, _IDXROWS - _NBUF)
      return x_hbm.at[pl.ds(idxrow0 + off, _NBUF)]

    def fire_idx(g, p):
      pltpu.async_copy(idx_src(g), idx_v.at[p], isem[p])

    def wait_idx(g, p):
      pltpu.make_async_copy(idx_src(g), idx_v.at[p], isem[p]).wait()

    def gather_b(g, b, p):
      pltpu.async_copy(
          tbl_v.at[idx_v.at[p].at[b]],
          rows_v.at[pl.ds(b * _SUB, _SUB)],
          gsem[b],
      )

    def wait_gather_b(g, b, p):
      pltpu.make_async_copy(
          tbl_v.at[idx_v.at[p].at[b]],
          rows_v.at[pl.ds(b * _SUB, _SUB)],
          gsem[b],
      ).wait()

    def scatter_b(g, b):
      pltpu.async_copy(
          rows_v.at[pl.ds(b * _SUB, _SUB)],
          out_hbm.at[pl.ds(outrow0 + g * _CHUNK + b * _SUB, _SUB)],
          ssem[b],
      )

    def wait_scatter_b(g, b):
      pltpu.make_async_copy(
          rows_v.at[pl.ds(b * _SUB, _SUB)],
          out_hbm.at[pl.ds(outrow0 + g * _CHUNK + b * _SUB, _SUB)],
          ssem[b],
      ).wait()

    def do_group(g, p, first):
      wait_idx(g, p)
      for b in range(_NBUF):
        if first is not True:
          wait_scatter_b(g - 1, b)
        gather_b(g, b, p)
      for b in range(_NBUF):
        wait_gather_b(g, b, p)
        scatter_b(g, b)
      if first is not False:
        fire_idx(g + 2, p)
      else:
        @pl.when(g + 2 < _NGRP)
        def _pf():
          fire_idx(g + 2, p)

    fire_idx(0, 0)
    fire_idx(1, 1)
    do_group(0, 0, True)
    do_group(1, 1, "peel")

    def step(i, carry):
      do_group(2 * i, 0, False)
      do_group(2 * i + 1, 1, False)
      return carry

    lax.fori_loop(1, _NGRP // 2, step, 0)
    for b in range(_NBUF):
      wait_scatter_b(_NGRP - 1, b)

  return body(x2d, emb_weight)


def kernel(x, emb_weight):
  x2d = x.reshape(_N // _SUB, _SUB).astype(jnp.int32)
  out = _emb_lookup(x2d, emb_weight)
  return out.reshape(_B, _T, _H)
